# baseline (speedup 1.0000x reference)
import functools

import jax
import jax.numpy as jnp
from jax import lax
from jax.experimental import pallas as pl
from jax.experimental.pallas import tpu as pltpu
from jax.experimental.pallas import tpu_sc as plsc

NUM = 1000000
B = 16384
D = 32
H = 64
PK = 4
GR = NUM // PK

_NC = 2
_NS = 16
_NW = _NC * _NS
_BPW = B // _NW
_CHUNK = 128
_NCHUNK = _BPW // _CHUNK

_TPC = 4096


def _tpk_body(u_ref, i_ref, ou_ref, oi_ref):
    def pack(x):
        return x.reshape(D, _TPC // PK, PK).transpose(1, 2, 0).reshape(
            _TPC // PK, 128)
    ou_ref[...] = pack(u_ref[...])
    oi_ref[...] = pack(i_ref[...])


_tpk = pl.pallas_call(
    _tpk_body,
    grid=(pl.cdiv(NUM, _TPC),),
    in_specs=[
        pl.BlockSpec((D, _TPC), lambda i: (0, i)),
        pl.BlockSpec((D, _TPC), lambda i: (0, i)),
    ],
    out_specs=[
        pl.BlockSpec((_TPC // PK, 128), lambda i: (i, 0)),
        pl.BlockSpec((_TPC // PK, 128), lambda i: (i, 0)),
    ],
    out_shape=(
        jax.ShapeDtypeStruct((GR, 128), jnp.float32),
        jax.ShapeDtypeStruct((GR, 128), jnp.float32),
    ),
)


@functools.partial(
    pl.kernel,
    out_type=(
        pltpu.HBM((B, 128), jnp.float32),
        pltpu.HBM((B, 128), jnp.float32),
    ),
    mesh=plsc.VectorSubcoreMesh(core_axis_name="c", subcore_axis_name="s"),
    scratch_types=[
        pltpu.VMEM((_NCHUNK, _CHUNK), jnp.int32),
        pltpu.VMEM((_NCHUNK, _CHUNK), jnp.int32),
        pltpu.VMEM((_BPW // 2, 128), jnp.float32),
        pltpu.VMEM((_BPW // 2, 128), jnp.float32),
        pltpu.SemaphoreType.DMA,
        pltpu.SemaphoreType.DMA,
    ],
)
def _sc_gather(uid2d, iid2d, utab, itab, out_u, out_i,
               idx_u, idx_i, rows_u, rows_i, sem_u, sem_i):
    wid = lax.axis_index("s") * _NC + lax.axis_index("c")
    base = wid * _BPW
    pltpu.sync_copy(uid2d.at[pl.ds(wid * _NCHUNK, _NCHUNK)], idx_u)
    pltpu.sync_copy(iid2d.at[pl.ds(wid * _NCHUNK, _NCHUNK)], idx_i)
    for j in range(_NCHUNK):
        for v in range(_CHUNK // 16):
            sl = pl.ds(v * 16, 16)
            idx_u[j, sl] = lax.shift_right_logical(idx_u[j, sl], 2)
            idx_i[j, sl] = lax.shift_right_logical(idx_i[j, sl], 2)
    for half in range(2):
        copies = []
        for j in range(_NCHUNK // 2):
            jj = half * (_NCHUNK // 2) + j
            copies.append(pltpu.async_copy(
                utab.at[idx_u.at[jj]],
                rows_u.at[pl.ds(j * _CHUNK, _CHUNK)], sem_u))
            copies.append(pltpu.async_copy(
                itab.at[idx_i.at[jj]],
                rows_i.at[pl.ds(j * _CHUNK, _CHUNK)], sem_i))
        for c in copies:
            c.wait()
        pltpu.sync_copy(rows_u, out_u.at[pl.ds(base + half * (_BPW // 2),
                                               _BPW // 2)])
        pltpu.sync_copy(rows_i, out_i.at[pl.ds(base + half * (_BPW // 2),
                                               _BPW // 2)])


_BLK = 2048


def _mlp_body(xu_ref, xi_ref, selu_ref, seli_ref, wu_ref, wi_ref, p_ref,
              out_ref):
    hu = jnp.dot(xu_ref[...], wu_ref[...], preferred_element_type=jnp.float32)
    hi = jnp.dot(xi_ref[...], wi_ref[...], preferred_element_type=jnp.float32)
    h = p_ref[0:1, :]
    for v in range(PK):
        h = h + hu[:, v * H:(v + 1) * H] * selu_ref[:, v:v + 1]
        h = h + hi[:, v * H:(v + 1) * H] * seli_ref[:, v:v + 1]
    h = jnp.maximum(h, 0.0)
    out_ref[...] = (jnp.sum(h * p_ref[1:2, :], axis=1, keepdims=True)
                    + p_ref[2:3, 0:1])


_mlp = pl.pallas_call(
    _mlp_body,
    grid=(B // _BLK,),
    in_specs=[
        pl.BlockSpec((_BLK, 128), lambda i: (i, 0)),
        pl.BlockSpec((_BLK, 128), lambda i: (i, 0)),
        pl.BlockSpec((_BLK, PK), lambda i: (i, 0)),
        pl.BlockSpec((_BLK, PK), lambda i: (i, 0)),
        pl.BlockSpec((128, PK * H), lambda i: (0, 0)),
        pl.BlockSpec((128, PK * H), lambda i: (0, 0)),
        pl.BlockSpec((8, H), lambda i: (0, 0)),
    ],
    out_specs=pl.BlockSpec((_BLK, 1), lambda i: (i, 0)),
    out_shape=jax.ShapeDtypeStruct((B, 1), jnp.float32),
)


def _shifted_weights(w_half):
    cols = []
    for v in range(PK):
        wv = jnp.zeros((128, H), jnp.float32)
        wv = wv.at[v * D:(v + 1) * D, :].set(w_half.T)
        cols.append(wv)
    return jnp.concatenate(cols, axis=1)


def kernel(user_ids, item_ids, user_table, item_table, W1, b1, Wf, bf):
    utg, itg = _tpk(user_table.T, item_table.T)
    uid2d = user_ids.reshape(B // _CHUNK, _CHUNK)
    iid2d = item_ids.reshape(B // _CHUNK, _CHUNK)
    xg_u, xg_i = _sc_gather(uid2d, iid2d, utg, itg)
    wu = _shifted_weights(W1[:, :D])
    wi = _shifted_weights(W1[:, D:])
    selu = (user_ids[:, None] % PK == jnp.arange(PK)[None, :]).astype(
        jnp.float32)
    seli = (item_ids[:, None] % PK == jnp.arange(PK)[None, :]).astype(
        jnp.float32)
    params = jnp.concatenate(
        [b1.reshape(1, H), Wf.reshape(1, H),
         jnp.broadcast_to(bf.reshape(1, 1), (1, H)),
         jnp.zeros((5, H), jnp.float32)], axis=0)
    return _mlp(xg_u, xg_i, selu, seli, wu, wi, params)


# speedup vs baseline: 8.7572x; 8.7572x over previous
import functools

import jax
import jax.numpy as jnp
from jax import lax
from jax.experimental import pallas as pl
from jax.experimental.pallas import tpu as pltpu
from jax.experimental.pallas import tpu_sc as plsc

NUM = 1000000
B = 16384
D = 32
H = 64
PK = 4
_TPC = 4096
_NBLK = pl.cdiv(NUM, _TPC)
_QB = _TPC // PK
GR = _NBLK * _QB

_NC = 2
_NS = 16
_NW = _NC * _NS
_BPW = B // _NW
_CHUNK = 128
_NCHUNK = _BPW // _CHUNK

def _tpk_body(u_ref, i_ref, ou_ref, oi_ref):
    def pack(x):
        t = x.T
        return jnp.concatenate(
            [t[k * _QB:(k + 1) * _QB] for k in range(PK)], axis=1)
    ou_ref[...] = pack(u_ref[...])
    oi_ref[...] = pack(i_ref[...])


_tpk = pl.pallas_call(
    _tpk_body,
    grid=(_NBLK,),
    in_specs=[
        pl.BlockSpec((D, _TPC), lambda i: (0, i)),
        pl.BlockSpec((D, _TPC), lambda i: (0, i)),
    ],
    out_specs=[
        pl.BlockSpec((_QB, 128), lambda i: (i, 0)),
        pl.BlockSpec((_QB, 128), lambda i: (i, 0)),
    ],
    out_shape=(
        jax.ShapeDtypeStruct((GR, 128), jnp.float32),
        jax.ShapeDtypeStruct((GR, 128), jnp.float32),
    ),
)


@functools.partial(
    pl.kernel,
    out_type=(
        pltpu.HBM((B, 128), jnp.float32),
        pltpu.HBM((B, 128), jnp.float32),
    ),
    mesh=plsc.VectorSubcoreMesh(core_axis_name="c", subcore_axis_name="s"),
    scratch_types=[
        pltpu.VMEM((_NCHUNK, _CHUNK), jnp.int32),
        pltpu.VMEM((_NCHUNK, _CHUNK), jnp.int32),
        pltpu.VMEM((_BPW // 2, 128), jnp.float32),
        pltpu.VMEM((_BPW // 2, 128), jnp.float32),
        pltpu.SemaphoreType.DMA,
        pltpu.SemaphoreType.DMA,
    ],
)
def _sc_gather(uid2d, iid2d, utab, itab, out_u, out_i,
               idx_u, idx_i, rows_u, rows_i, sem_u, sem_i):
    wid = lax.axis_index("s") * _NC + lax.axis_index("c")
    base = wid * _BPW
    pltpu.sync_copy(uid2d.at[pl.ds(wid * _NCHUNK, _NCHUNK)], idx_u)
    pltpu.sync_copy(iid2d.at[pl.ds(wid * _NCHUNK, _NCHUNK)], idx_i)
    for j in range(_NCHUNK):
        for v in range(_CHUNK // 16):
            sl = pl.ds(v * 16, 16)
            xu = idx_u[j, sl]
            idx_u[j, sl] = (lax.shift_left(
                lax.shift_right_logical(xu, 12), 10)
                + lax.bitwise_and(xu, 1023))
            xi = idx_i[j, sl]
            idx_i[j, sl] = (lax.shift_left(
                lax.shift_right_logical(xi, 12), 10)
                + lax.bitwise_and(xi, 1023))
    for half in range(2):
        copies = []
        for j in range(_NCHUNK // 2):
            jj = half * (_NCHUNK // 2) + j
            copies.append(pltpu.async_copy(
                utab.at[idx_u.at[jj]],
                rows_u.at[pl.ds(j * _CHUNK, _CHUNK)], sem_u))
            copies.append(pltpu.async_copy(
                itab.at[idx_i.at[jj]],
                rows_i.at[pl.ds(j * _CHUNK, _CHUNK)], sem_i))
        for c in copies:
            c.wait()
        pltpu.sync_copy(rows_u, out_u.at[pl.ds(base + half * (_BPW // 2),
                                               _BPW // 2)])
        pltpu.sync_copy(rows_i, out_i.at[pl.ds(base + half * (_BPW // 2),
                                               _BPW // 2)])


_BLK = 2048


def _mlp_body(xu_ref, xi_ref, selu_ref, seli_ref, wu_ref, wi_ref, p_ref,
              out_ref):
    hu = jnp.dot(xu_ref[...], wu_ref[...], preferred_element_type=jnp.float32)
    hi = jnp.dot(xi_ref[...], wi_ref[...], preferred_element_type=jnp.float32)
    h = p_ref[0:1, :]
    for v in range(PK):
        h = h + hu[:, v * H:(v + 1) * H] * selu_ref[:, v:v + 1]
        h = h + hi[:, v * H:(v + 1) * H] * seli_ref[:, v:v + 1]
    h = jnp.maximum(h, 0.0)
    out_ref[...] = (jnp.sum(h * p_ref[1:2, :], axis=1, keepdims=True)
                    + p_ref[2:3, 0:1])


_mlp = pl.pallas_call(
    _mlp_body,
    grid=(B // _BLK,),
    in_specs=[
        pl.BlockSpec((_BLK, 128), lambda i: (i, 0)),
        pl.BlockSpec((_BLK, 128), lambda i: (i, 0)),
        pl.BlockSpec((_BLK, PK), lambda i: (i, 0)),
        pl.BlockSpec((_BLK, PK), lambda i: (i, 0)),
        pl.BlockSpec((128, PK * H), lambda i: (0, 0)),
        pl.BlockSpec((128, PK * H), lambda i: (0, 0)),
        pl.BlockSpec((8, H), lambda i: (0, 0)),
    ],
    out_specs=pl.BlockSpec((_BLK, 1), lambda i: (i, 0)),
    out_shape=jax.ShapeDtypeStruct((B, 1), jnp.float32),
)


def _shifted_weights(w_half):
    cols = []
    for v in range(PK):
        wv = jnp.zeros((128, H), jnp.float32)
        wv = wv.at[v * D:(v + 1) * D, :].set(w_half.T)
        cols.append(wv)
    return jnp.concatenate(cols, axis=1)


def kernel(user_ids, item_ids, user_table, item_table, W1, b1, Wf, bf):
    utg, itg = _tpk(user_table.T, item_table.T)
    uid2d = user_ids.reshape(B // _CHUNK, _CHUNK)
    iid2d = item_ids.reshape(B // _CHUNK, _CHUNK)
    xg_u, xg_i = _sc_gather(uid2d, iid2d, utg, itg)
    wu = _shifted_weights(W1[:, :D])
    wi = _shifted_weights(W1[:, D:])
    ku = (user_ids[:, None] >> 10) & 3
    ki = (item_ids[:, None] >> 10) & 3
    selu = (ku == jnp.arange(PK)[None, :]).astype(jnp.float32)
    seli = (ki == jnp.arange(PK)[None, :]).astype(jnp.float32)
    params = jnp.concatenate(
        [b1.reshape(1, H), Wf.reshape(1, H),
         jnp.broadcast_to(bf.reshape(1, 1), (1, H)),
         jnp.zeros((5, H), jnp.float32)], axis=0)
    return _mlp(xg_u, xg_i, selu, seli, wu, wi, params)


# speedup vs baseline: 10.9453x; 1.2499x over previous
import functools

import jax
import jax.numpy as jnp
from jax import lax
from jax.experimental import pallas as pl
from jax.experimental.pallas import tpu as pltpu
from jax.experimental.pallas import tpu_sc as plsc

NUM = 1000000
B = 16384
D = 32
H = 64

_NC = 2
_NS = 16
_NW = _NC * _NS
_BPW = B // _NW
_CHUNK = 128
_NCHUNK = _BPW // _CHUNK

_TPC = 8192


def _tpk_body(u_ref, i_ref, ou_ref, oi_ref):
    z = jnp.zeros((_TPC, 128 - D), jnp.float32)
    ou_ref[...] = jnp.concatenate([u_ref[...].T, z], axis=1)
    oi_ref[...] = jnp.concatenate([i_ref[...].T, z], axis=1)


_tpk = pl.pallas_call(
    _tpk_body,
    grid=(pl.cdiv(NUM, _TPC),),
    in_specs=[
        pl.BlockSpec((D, _TPC), lambda i: (0, i)),
        pl.BlockSpec((D, _TPC), lambda i: (0, i)),
    ],
    out_specs=[
        pl.BlockSpec((_TPC, 128), lambda i: (i, 0)),
        pl.BlockSpec((_TPC, 128), lambda i: (i, 0)),
    ],
    out_shape=(
        jax.ShapeDtypeStruct((NUM, 128), jnp.float32),
        jax.ShapeDtypeStruct((NUM, 128), jnp.float32),
    ),
)


@functools.partial(
    pl.kernel,
    out_type=(
        pltpu.HBM((B, 128), jnp.float32),
        pltpu.HBM((B, 128), jnp.float32),
    ),
    mesh=plsc.VectorSubcoreMesh(core_axis_name="c", subcore_axis_name="s"),
    scratch_types=[
        pltpu.VMEM((_NCHUNK, _CHUNK), jnp.int32),
        pltpu.VMEM((_NCHUNK, _CHUNK), jnp.int32),
        pltpu.VMEM((_BPW // 2, 128), jnp.float32),
        pltpu.VMEM((_BPW // 2, 128), jnp.float32),
        pltpu.SemaphoreType.DMA,
        pltpu.SemaphoreType.DMA,
    ],
)
def _sc_gather(uid2d, iid2d, utab, itab, out_u, out_i,
               idx_u, idx_i, rows_u, rows_i, sem_u, sem_i):
    wid = lax.axis_index("s") * _NC + lax.axis_index("c")
    base = wid * _BPW
    pltpu.sync_copy(uid2d.at[pl.ds(wid * _NCHUNK, _NCHUNK)], idx_u)
    pltpu.sync_copy(iid2d.at[pl.ds(wid * _NCHUNK, _NCHUNK)], idx_i)
    for half in range(2):
        copies = []
        for j in range(_NCHUNK // 2):
            jj = half * (_NCHUNK // 2) + j
            copies.append(pltpu.async_copy(
                utab.at[idx_u.at[jj]],
                rows_u.at[pl.ds(j * _CHUNK, _CHUNK)], sem_u))
            copies.append(pltpu.async_copy(
                itab.at[idx_i.at[jj]],
                rows_i.at[pl.ds(j * _CHUNK, _CHUNK)], sem_i))
        for c in copies:
            c.wait()
        pltpu.sync_copy(rows_u, out_u.at[pl.ds(base + half * (_BPW // 2),
                                               _BPW // 2)])
        pltpu.sync_copy(rows_i, out_i.at[pl.ds(base + half * (_BPW // 2),
                                               _BPW // 2)])


_BLK = 2048


def _mlp_body(xu_ref, xi_ref, wu_ref, wi_ref, p_ref, out_ref):
    h = jnp.dot(xu_ref[...], wu_ref[...], preferred_element_type=jnp.float32)
    h = h + jnp.dot(xi_ref[...], wi_ref[...],
                    preferred_element_type=jnp.float32)
    h = jnp.maximum(h + p_ref[0:1, :], 0.0)
    out_ref[...] = (jnp.sum(h * p_ref[1:2, :], axis=1, keepdims=True)
                    + p_ref[2:3, 0:1])


_mlp = pl.pallas_call(
    _mlp_body,
    grid=(B // _BLK,),
    in_specs=[
        pl.BlockSpec((_BLK, 128), lambda i: (i, 0)),
        pl.BlockSpec((_BLK, 128), lambda i: (i, 0)),
        pl.BlockSpec((128, H), lambda i: (0, 0)),
        pl.BlockSpec((128, H), lambda i: (0, 0)),
        pl.BlockSpec((8, H), lambda i: (0, 0)),
    ],
    out_specs=pl.BlockSpec((_BLK, 1), lambda i: (i, 0)),
    out_shape=jax.ShapeDtypeStruct((B, 1), jnp.float32),
)


def kernel(user_ids, item_ids, user_table, item_table, W1, b1, Wf, bf):
    utg, itg = _tpk(user_table.T, item_table.T)
    uid2d = user_ids.reshape(B // _CHUNK, _CHUNK)
    iid2d = item_ids.reshape(B // _CHUNK, _CHUNK)
    xg_u, xg_i = _sc_gather(uid2d, iid2d, utg, itg)
    wu = jnp.zeros((128, H), jnp.float32).at[:D].set(W1[:, :D].T)
    wi = jnp.zeros((128, H), jnp.float32).at[:D].set(W1[:, D:].T)
    params = jnp.concatenate(
        [b1.reshape(1, H), Wf.reshape(1, H),
         jnp.broadcast_to(bf.reshape(1, 1), (1, H)),
         jnp.zeros((5, H), jnp.float32)], axis=0)
    return _mlp(xg_u, xg_i, wu, wi, params)


# speedup vs baseline: 11.2430x; 1.0272x over previous
import functools

import jax
import jax.numpy as jnp
from jax import lax
from jax.experimental import pallas as pl
from jax.experimental.pallas import tpu as pltpu
from jax.experimental.pallas import tpu_sc as plsc

NUM = 1000000
B = 16384
D = 32
H = 64

_NC = 2
_NS = 16
_NW = _NC * _NS
_BPW = B // _NW
_CHUNK = 128
_NCHUNK = _BPW // _CHUNK

_TPC = 16384


def _tpk_body(u_ref, i_ref, ou_ref, oi_ref):
    z = jnp.zeros((_TPC, 128 - D), jnp.float32)
    ou_ref[...] = jnp.concatenate([u_ref[...].T, z], axis=1)
    oi_ref[...] = jnp.concatenate([i_ref[...].T, z], axis=1)


_tpk = pl.pallas_call(
    _tpk_body,
    grid=(pl.cdiv(NUM, _TPC),),
    in_specs=[
        pl.BlockSpec((D, _TPC), lambda i: (0, i)),
        pl.BlockSpec((D, _TPC), lambda i: (0, i)),
    ],
    out_specs=[
        pl.BlockSpec((_TPC, 128), lambda i: (i, 0)),
        pl.BlockSpec((_TPC, 128), lambda i: (i, 0)),
    ],
    out_shape=(
        jax.ShapeDtypeStruct((NUM, 128), jnp.float32),
        jax.ShapeDtypeStruct((NUM, 128), jnp.float32),
    ),
)


@functools.partial(
    pl.kernel,
    out_type=(
        pltpu.HBM((B, 128), jnp.float32),
        pltpu.HBM((B, 128), jnp.float32),
    ),
    mesh=plsc.VectorSubcoreMesh(core_axis_name="c", subcore_axis_name="s"),
    scratch_types=[
        pltpu.VMEM((_NCHUNK, _CHUNK), jnp.int32),
        pltpu.VMEM((_NCHUNK, _CHUNK), jnp.int32),
        pltpu.VMEM((_BPW // 2, 128), jnp.float32),
        pltpu.VMEM((_BPW // 2, 128), jnp.float32),
        pltpu.SemaphoreType.DMA,
        pltpu.SemaphoreType.DMA,
    ],
)
def _sc_gather(uid2d, iid2d, utab, itab, out_u, out_i,
               idx_u, idx_i, rows_u, rows_i, sem_u, sem_i):
    wid = lax.axis_index("s") * _NC + lax.axis_index("c")
    base = wid * _BPW
    pltpu.sync_copy(uid2d.at[pl.ds(wid * _NCHUNK, _NCHUNK)], idx_u)
    pltpu.sync_copy(iid2d.at[pl.ds(wid * _NCHUNK, _NCHUNK)], idx_i)
    for half in range(2):
        copies = []
        for j in range(_NCHUNK // 2):
            jj = half * (_NCHUNK // 2) + j
            copies.append(pltpu.async_copy(
                utab.at[idx_u.at[jj]],
                rows_u.at[pl.ds(j * _CHUNK, _CHUNK)], sem_u))
            copies.append(pltpu.async_copy(
                itab.at[idx_i.at[jj]],
                rows_i.at[pl.ds(j * _CHUNK, _CHUNK)], sem_i))
        for c in copies:
            c.wait()
        pltpu.sync_copy(rows_u, out_u.at[pl.ds(base + half * (_BPW // 2),
                                               _BPW // 2)])
        pltpu.sync_copy(rows_i, out_i.at[pl.ds(base + half * (_BPW // 2),
                                               _BPW // 2)])


_BLK = 2048


def _mlp_body(xu_ref, xi_ref, wu_ref, wi_ref, p_ref, out_ref):
    h = jnp.dot(xu_ref[...], wu_ref[...], preferred_element_type=jnp.float32)
    h = h + jnp.dot(xi_ref[...], wi_ref[...],
                    preferred_element_type=jnp.float32)
    h = jnp.maximum(h + p_ref[0:1, :], 0.0)
    out_ref[...] = (jnp.sum(h * p_ref[1:2, :], axis=1, keepdims=True)
                    + p_ref[2:3, 0:1])


_mlp = pl.pallas_call(
    _mlp_body,
    grid=(B // _BLK,),
    in_specs=[
        pl.BlockSpec((_BLK, 128), lambda i: (i, 0)),
        pl.BlockSpec((_BLK, 128), lambda i: (i, 0)),
        pl.BlockSpec((128, H), lambda i: (0, 0)),
        pl.BlockSpec((128, H), lambda i: (0, 0)),
        pl.BlockSpec((8, H), lambda i: (0, 0)),
    ],
    out_specs=pl.BlockSpec((_BLK, 1), lambda i: (i, 0)),
    out_shape=jax.ShapeDtypeStruct((B, 1), jnp.float32),
)


def kernel(user_ids, item_ids, user_table, item_table, W1, b1, Wf, bf):
    utg, itg = _tpk(user_table.T, item_table.T)
    uid2d = user_ids.reshape(B // _CHUNK, _CHUNK)
    iid2d = item_ids.reshape(B // _CHUNK, _CHUNK)
    xg_u, xg_i = _sc_gather(uid2d, iid2d, utg, itg)
    wu = jnp.zeros((128, H), jnp.float32).at[:D].set(W1[:, :D].T)
    wi = jnp.zeros((128, H), jnp.float32).at[:D].set(W1[:, D:].T)
    params = jnp.concatenate(
        [b1.reshape(1, H), Wf.reshape(1, H),
         jnp.broadcast_to(bf.reshape(1, 1), (1, H)),
         jnp.zeros((5, H), jnp.float32)], axis=0)
    return _mlp(xg_u, xg_i, wu, wi, params)
